# baseline (speedup 1.0000x reference)
import jax
import jax.numpy as jnp
from jax import lax
from jax.experimental import pallas as pl
from jax.experimental.pallas import tpu as pltpu
from jax.experimental.pallas import tpu_sc as plsc

N = 10000
NP = 10240
B = 16
HID = 64
PADW = 80
E = 160000
NC = 2
NS = 16
EPT = E // NS
EPTP = 10240
G = 128
NBLK = EPTP // G
NSTG = 4
RPT = NP // NS
BPC = B // NC
NVR = PADW // 16


def _sc_body(x0_ref, c1_ref, r1_ref, v1_ref, c2_ref, r2_ref, v2_ref, z_ref,
             y1_ref, y2_ref, y3_ref, y4_ref,
             col_v, row_v, val_v, idx0, idx1, idx2, idx3,
             st0, st1, st2, st3, acc,
             gsem0, gsem1, gsem2, gsem3, ssem0, ssem1, ssem2, ssem3):
    c = lax.axis_index("c")
    s = lax.axis_index("s")
    stages = (st0, st1, st2, st3)
    idxs = (idx0, idx1, idx2, idx3)
    gsems = (gsem0, gsem1, gsem2, gsem3)
    ssems = (ssem0, ssem1, ssem2, ssem3)

    def mk_idx(p, k, off):
        for i in range(G // 16):
            sl = pl.ds(i * 16, 16)
            idxs[p][sl] = col_v[pl.ds(k * G + i * 16, 16)] + off

    def scale(p, kG):
        st = stages[p]

        def grp(g, carry):
            chunk = val_v[pl.ds(carry + g * 16, 16)]
            for u in range(16):
                vv = jnp.broadcast_to(chunk[u], (16,))
                j = g * 16 + u
                for r in range(NVR):
                    st[j, pl.ds(r * 16, 16)] = st[j, pl.ds(r * 16, 16)] * vv
            return carry
        lax.fori_loop(0, G // 16, grp, kG)

    def spmm_pass(src_ref, dst_ref, b):
        pltpu.sync_copy(z_ref, acc.at[pl.ds(s * RPT, RPT)])
        plsc.subcore_barrier()

        off = b * NP
        mk_idx(0, 0, off)
        pltpu.async_copy(src_ref.at[idx0], st0, gsem0)
        mk_idx(1, 1, off)
        pltpu.async_copy(src_ref.at[idx1], st1, gsem1)

        def blk(m, _):
            for u in range(NSTG):
                k = m * NSTG + u
                w = (u + 2) % NSTG

                pltpu.make_async_copy(src_ref.at[idxs[u]], stages[u],
                                      gsems[u]).wait()
                scale(u, k * G)
                pltpu.async_copy(stages[u], acc.at[row_v.at[k]], ssems[u],
                                 add=True)

                @pl.when(k + 2 < NBLK)
                def _prefetch():
                    mk_idx(w, k + 2, off)

                    @pl.when(k >= 2)
                    def _drain_prev_scatter():
                        pltpu.make_async_copy(
                            stages[w], acc.at[row_v.at[k - 2]],
                            ssems[w]).wait()
                    pltpu.async_copy(src_ref.at[idxs[w]], stages[w],
                                     gsems[w])
            return 0
        lax.fori_loop(0, NBLK // NSTG, blk, 0)
        for i in range(NSTG):
            kk = NBLK - NSTG + i
            pltpu.make_async_copy(
                stages[kk % NSTG], acc.at[row_v.at[kk]],
                ssems[kk % NSTG]).wait()
        plsc.subcore_barrier()
        pltpu.sync_copy(acc.at[pl.ds(s * RPT, RPT)],
                        dst_ref.at[pl.ds(b * NP + s * RPT, RPT)])

    for (ch, rh, vh, dst_a, dst_b) in (
            (c1_ref, r1_ref, v1_ref, y1_ref, y2_ref),
            (c2_ref, r2_ref, v2_ref, y3_ref, y4_ref)):
        pltpu.sync_copy(ch.at[s], col_v)
        pltpu.sync_copy(rh.at[s], row_v)
        pltpu.sync_copy(vh.at[s], val_v)

        def batch_body(bi, _):
            b = c * BPC + bi
            spmm_pass(x0_ref, dst_a, b)
            spmm_pass(dst_a, dst_b, b)
            return 0
        lax.fori_loop(0, BPC, batch_body, 0)


def _mm_body(x0_ref, y1_ref, y2_ref, y3_ref, y4_ref, w_ref, b_ref, o_ref):
    acc = jnp.dot(x0_ref[0], w_ref[0], preferred_element_type=jnp.float32)
    acc += jnp.dot(y1_ref[0], w_ref[1], preferred_element_type=jnp.float32)
    acc += jnp.dot(y2_ref[0], w_ref[2], preferred_element_type=jnp.float32)
    acc += jnp.dot(y3_ref[0], w_ref[3], preferred_element_type=jnp.float32)
    acc += jnp.dot(y4_ref[0], w_ref[4], preferred_element_type=jnp.float32)
    o_ref[0] = jnp.tanh(acc + b_ref[...])


def _prep_edges(col, row, val):
    perm = jnp.argsort(col)
    col, row, val = col[perm], row[perm], val[perm]
    cp = jnp.pad(col.reshape(NS, EPT), ((0, 0), (0, EPTP - EPT)))
    rp = jnp.pad(row.reshape(NS, EPT), ((0, 0), (0, EPTP - EPT)))
    vp = jnp.pad(val.reshape(NS, EPT), ((0, 0), (0, EPTP - EPT)))
    return cp, rp.reshape(NS, NBLK, G), vp


def kernel(inputs, state_t, s1_row, s1_col, s1_val, s2_row, s2_col, s2_val,
           weights, biases):
    Bb, Nn, in_dim = inputs.shape
    x_cat = jnp.concatenate([inputs, state_t], axis=2)
    in_size = x_cat.shape[2]
    x0p = jnp.pad(x_cat, ((0, 0), (0, NP - Nn), (0, PADW - in_size)))
    x0f = x0p.reshape(Bb * NP, PADW)
    zeros_hbm = jnp.zeros((RPT, PADW), jnp.float32)

    c1, r1, v1 = _prep_edges(s1_col, s1_row, s1_val)
    c2, r2, v2 = _prep_edges(s2_col, s2_row, s2_val)

    mesh = plsc.VectorSubcoreMesh(core_axis_name="c", subcore_axis_name="s")
    sc = pl.kernel(
        _sc_body,
        out_type=[jax.ShapeDtypeStruct((Bb * NP, PADW), jnp.float32)] * 4,
        mesh=mesh,
        compiler_params=pltpu.CompilerParams(use_tc_tiling_on_sc=False),
        scratch_types=[
            pltpu.VMEM((EPTP,), jnp.int32),
            pltpu.VMEM((NBLK, G), jnp.int32),
            pltpu.VMEM((EPTP,), jnp.float32),
            pltpu.VMEM((G,), jnp.int32),
            pltpu.VMEM((G,), jnp.int32),
            pltpu.VMEM((G,), jnp.int32),
            pltpu.VMEM((G,), jnp.int32),
            pltpu.VMEM((G, PADW), jnp.float32),
            pltpu.VMEM((G, PADW), jnp.float32),
            pltpu.VMEM((G, PADW), jnp.float32),
            pltpu.VMEM((G, PADW), jnp.float32),
            pltpu.VMEM_SHARED((NP, PADW), jnp.float32),
        ] + [pltpu.SemaphoreType.DMA] * 8,
    )
    y1, y2, y3, y4 = sc(x0f, c1, r1, v1, c2, r2, v2, zeros_hbm)

    wm = weights.reshape(in_size, 5, HID)
    wa = jnp.stack([wm[:, 0] - wm[:, 2] - wm[:, 4], wm[:, 1], 2.0 * wm[:, 2],
                    wm[:, 3], 2.0 * wm[:, 4]], axis=0)
    wp = jnp.pad(wa, ((0, 0), (0, PADW - in_size), (0, 0)))

    NB = 1000
    feat_spec = pl.BlockSpec((1, NB, PADW), lambda bb, nn: (bb, nn, 0))
    out = pl.pallas_call(
        _mm_body,
        grid=(Bb, Nn // NB),
        in_specs=[feat_spec] * 5 + [
            pl.BlockSpec((5, PADW, HID), lambda bb, nn: (0, 0, 0)),
            pl.BlockSpec((HID,), lambda bb, nn: (0,)),
        ],
        out_specs=pl.BlockSpec((1, NB, HID), lambda bb, nn: (bb, nn, 0)),
        out_shape=jax.ShapeDtypeStruct((Bb, Nn, HID), jnp.float32),
    )(x0p, y1.reshape(Bb, NP, PADW), y2.reshape(Bb, NP, PADW),
      y3.reshape(Bb, NP, PADW), y4.reshape(Bb, NP, PADW), wp, biases)
    return out


# speedup vs baseline: 1.2123x; 1.2123x over previous
import jax
import jax.numpy as jnp
from jax import lax
from jax.experimental import pallas as pl
from jax.experimental.pallas import tpu as pltpu
from jax.experimental.pallas import tpu_sc as plsc

N = 10000
NP = 10240
B = 16
NPAIR = B // 2
HID = 64
PADW = 80
W2 = 2 * PADW
E = 160000
NC = 2
NS = 16
EPT = E // NS
EPTP = 10240
G = 64
NBLK = EPTP // G
NSTG = 2
NEB = 4
RPT = NP // NS
PPC = NPAIR // NC
NVR = W2 // 16


def _sc_body(x0_ref, e1_ref, v1_ref, e2_ref, v2_ref, z_ref,
             y1_ref, y2_ref, y3_ref, y4_ref,
             eb0, eb1, eb2, eb3, vb0, vb1, vb2, vb3,
             idx0, idx1, st0, st1, acc,
             esem0, esem1, esem2, esem3,
             vsem0, vsem1, vsem2, vsem3, gsem0, gsem1, ssem0, ssem1):
    c = lax.axis_index("c")
    s = lax.axis_index("s")
    ebufs = (eb0, eb1, eb2, eb3)
    vbufs = (vb0, vb1, vb2, vb3)
    esems = (esem0, esem1, esem2, esem3)
    vsems = (vsem0, vsem1, vsem2, vsem3)
    idxs = (idx0, idx1)
    stages = (st0, st1)
    gsems = (gsem0, gsem1)
    ssems = (ssem0, ssem1)

    def ld_edges(e_ref, v_ref, n, k):
        pltpu.async_copy(v_ref.at[s * NBLK + k], vbufs[n], vsems[n])
        return pltpu.async_copy(e_ref.at[s * NBLK + k], ebufs[n], esems[n])

    def mk_idx(p, n, off):
        for i in range(G // 16):
            sl = pl.ds(i * 16, 16)
            idxs[p][sl] = ebufs[n][0, sl] + off

    def scale(p, n):
        st = stages[p]
        eb = ebufs[n]

        vb = vbufs[n]

        def grp(g, carry):
            vf = vb[pl.ds(g * 16, 16)]
            for u in range(16):
                vv = jnp.broadcast_to(vf[u], (16,))
                j = g * 16 + u
                for r in range(NVR):
                    st[j, pl.ds(r * 16, 16)] = st[j, pl.ds(r * 16, 16)] * vv
            return carry
        lax.fori_loop(0, G // 16, grp, 0)

    def spmm_pass(e_ref, v_ref, src_ref, dst_ref, pr):
        pltpu.sync_copy(z_ref, acc.at[pl.ds(s * RPT, RPT)])
        plsc.subcore_barrier()

        off = pr * NP
        ld_edges(e_ref, v_ref, 0, 0).wait()
        pltpu.make_async_copy(v_ref.at[s * NBLK], vbufs[0], vsems[0]).wait()
        ld_edges(e_ref, v_ref, 1, 1)
        ld_edges(e_ref, v_ref, 2, 2)
        mk_idx(0, 0, off)
        pltpu.async_copy(src_ref.at[idx0], st0, gsem0)

        def blk(m, _):
            for u in range(NSTG * 2):
                k = m * NSTG * 2 + u
                p = u % NSTG
                q = (u + 1) % NSTG
                n = u % NEB
                n1 = (u + 1) % NEB
                n3 = (u + 3) % NEB

                @pl.when(k + 1 < NBLK)
                def _prefetch_gather():
                    pltpu.make_async_copy(e_ref.at[s * NBLK + k + 1],
                                          ebufs[n1], esems[n1]).wait()
                    pltpu.make_async_copy(v_ref.at[s * NBLK + k + 1],
                                          vbufs[n1], vsems[n1]).wait()
                    mk_idx(q, n1, off)

                    @pl.when(k >= 1)
                    def _drain_prev_scatter():
                        pltpu.make_async_copy(
                            stages[q], acc.at[ebufs[n3].at[1]],
                            ssems[q]).wait()
                    pltpu.async_copy(src_ref.at[idxs[q]], stages[q],
                                     gsems[q])

                @pl.when(k + 3 < NBLK)
                def _prefetch_edges():
                    ld_edges(e_ref, v_ref, n3, k + 3)

                pltpu.make_async_copy(src_ref.at[idxs[p]], stages[p],
                                      gsems[p]).wait()
                scale(p, n)
                pltpu.async_copy(stages[p], acc.at[ebufs[n].at[1]], ssems[p],
                                 add=True)
            return 0
        lax.fori_loop(0, NBLK // (NSTG * 2), blk, 0)
        for i in range(NSTG):
            kk = NBLK - NSTG + i
            pltpu.make_async_copy(
                stages[kk % NSTG], acc.at[ebufs[kk % NEB].at[1]],
                ssems[kk % NSTG]).wait()
        plsc.subcore_barrier()
        pltpu.sync_copy(acc.at[pl.ds(s * RPT, RPT)],
                        dst_ref.at[pl.ds(pr * NP + s * RPT, RPT)])

    for (eh, vh, dst_a, dst_b) in ((e1_ref, v1_ref, y1_ref, y2_ref),
                                   (e2_ref, v2_ref, y3_ref, y4_ref)):
        def pair_body(bi, _):
            pr = c * PPC + bi
            spmm_pass(eh, vh, x0_ref, dst_a, pr)
            spmm_pass(eh, vh, dst_a, dst_b, pr)
            return 0
        lax.fori_loop(0, PPC, pair_body, 0)


def _mm_body(x0_ref, y1_ref, y2_ref, y3_ref, y4_ref, w_ref, b_ref, o_ref):
    for h in range(2):
        acc = jnp.dot(x0_ref[0, :, h], w_ref[0],
                      preferred_element_type=jnp.float32)
        acc += jnp.dot(y1_ref[0, :, h], w_ref[1],
                       preferred_element_type=jnp.float32)
        acc += jnp.dot(y2_ref[0, :, h], w_ref[2],
                       preferred_element_type=jnp.float32)
        acc += jnp.dot(y3_ref[0, :, h], w_ref[3],
                       preferred_element_type=jnp.float32)
        acc += jnp.dot(y4_ref[0, :, h], w_ref[4],
                       preferred_element_type=jnp.float32)
        o_ref[0, h] = jnp.tanh(acc + b_ref[...])


def _prep_edges(col, row, val):
    cp = jnp.pad(col.reshape(NS, EPT), ((0, 0), (0, EPTP - EPT)))
    rp = jnp.pad(row.reshape(NS, EPT), ((0, 0), (0, EPTP - EPT)))
    vp = jnp.pad(val.reshape(NS, EPT), ((0, 0), (0, EPTP - EPT)))
    packed = jnp.stack([cp.reshape(NS, NBLK, G), rp.reshape(NS, NBLK, G)],
                       axis=2)
    return packed.reshape(NS * NBLK, 2, G), vp.reshape(NS * NBLK, G)


def kernel(inputs, state_t, s1_row, s1_col, s1_val, s2_row, s2_col, s2_val,
           weights, biases):
    Bb, Nn, in_dim = inputs.shape
    x_cat = jnp.concatenate([inputs, state_t], axis=2)
    in_size = x_cat.shape[2]
    x0p = jnp.pad(x_cat, ((0, 0), (0, NP - Nn), (0, PADW - in_size)))
    x0pair = jnp.transpose(x0p.reshape(NPAIR, 2, NP, PADW),
                           (0, 2, 1, 3)).reshape(NPAIR * NP, W2)
    zeros_hbm = jnp.zeros((RPT, W2), jnp.float32)

    e1, v1 = _prep_edges(s1_col, s1_row, s1_val)
    e2, v2 = _prep_edges(s2_col, s2_row, s2_val)

    mesh = plsc.VectorSubcoreMesh(core_axis_name="c", subcore_axis_name="s")
    sc = pl.kernel(
        _sc_body,
        out_type=[jax.ShapeDtypeStruct((NPAIR * NP, W2), jnp.float32)] * 4,
        mesh=mesh,
        compiler_params=pltpu.CompilerParams(use_tc_tiling_on_sc=False),
        scratch_types=[
            pltpu.VMEM((2, G), jnp.int32),
            pltpu.VMEM((2, G), jnp.int32),
            pltpu.VMEM((2, G), jnp.int32),
            pltpu.VMEM((2, G), jnp.int32),
            pltpu.VMEM((G,), jnp.float32),
            pltpu.VMEM((G,), jnp.float32),
            pltpu.VMEM((G,), jnp.float32),
            pltpu.VMEM((G,), jnp.float32),
            pltpu.VMEM((G,), jnp.int32),
            pltpu.VMEM((G,), jnp.int32),
            pltpu.VMEM((G, W2), jnp.float32),
            pltpu.VMEM((G, W2), jnp.float32),
            pltpu.VMEM_SHARED((NP, W2), jnp.float32),
        ] + [pltpu.SemaphoreType.DMA] * 12,
    )
    y1, y2, y3, y4 = sc(x0pair, e1, v1, e2, v2, zeros_hbm)

    wm = weights.reshape(in_size, 5, HID)
    wa = jnp.stack([wm[:, 0] - wm[:, 2] - wm[:, 4], wm[:, 1], 2.0 * wm[:, 2],
                    wm[:, 3], 2.0 * wm[:, 4]], axis=0)
    wp = jnp.pad(wa, ((0, 0), (0, PADW - in_size), (0, 0)))

    NB = 1000
    feat_spec = pl.BlockSpec((1, NB, 2, PADW), lambda pp, nn: (pp, nn, 0, 0))
    fshape = (NPAIR, NP, 2, PADW)
    out = pl.pallas_call(
        _mm_body,
        grid=(NPAIR, Nn // NB),
        in_specs=[feat_spec] * 5 + [
            pl.BlockSpec((5, PADW, HID), lambda pp, nn: (0, 0, 0)),
            pl.BlockSpec((HID,), lambda pp, nn: (0,)),
        ],
        out_specs=pl.BlockSpec((1, 2, NB, HID), lambda pp, nn: (pp, 0, nn, 0)),
        out_shape=jax.ShapeDtypeStruct((NPAIR, 2, Nn, HID), jnp.float32),
    )(x0pair.reshape(fshape), y1.reshape(fshape), y2.reshape(fshape),
      y3.reshape(fshape), y4.reshape(fshape), wp, biases)
    return out.reshape(Bb, Nn, HID)


# speedup vs baseline: 1.2843x; 1.0594x over previous
import jax
import jax.numpy as jnp
from jax import lax
from jax.experimental import pallas as pl
from jax.experimental.pallas import tpu as pltpu
from jax.experimental.pallas import tpu_sc as plsc

N = 10000
NP = 10240
B = 16
HID = 64
PADW = 80
E = 160000
NC = 2
NS = 16
EPT = E // NS
EPTP = 10240
G = 256
GD = 128
NBLK = EPTP // G
NSTG = 2
RPT = NP // NS
BPC = B // NC
NVR = PADW // 16


def _sc_body(x0_ref, c1_ref, r1_ref, v1_ref, c2_ref, r2_ref, v2_ref, z_ref,
             y1_ref, y2_ref, y3_ref, y4_ref,
             col_v, row_v, val_v, idx0, idx1,
             st0, st1, acc,
             gsem0, gsem1, ssem0, ssem1):
    c = lax.axis_index("c")
    s = lax.axis_index("s")
    stages = (st0, st1)
    idxs = (idx0, idx1)
    gsems = (gsem0, gsem1)
    ssems = (ssem0, ssem1)

    def mk_idx(p, k, off):
        for i in range(G // 16):
            sl = pl.ds(i * 16, 16)
            idxs[p][sl] = col_v[pl.ds(k * G + i * 16, 16)] + off

    def scale(p, kG):
        st = stages[p]

        def grp(g, carry):
            chunk = val_v[pl.ds(carry + g * 16, 16)]
            for u in range(16):
                vv = jnp.broadcast_to(chunk[u], (16,))
                j = g * 16 + u
                for r in range(NVR):
                    st[j, pl.ds(r * 16, 16)] = st[j, pl.ds(r * 16, 16)] * vv
            return carry
        lax.fori_loop(0, G // 16, grp, kG)

    def fire_gather(p, src_ref):
        for d in range(G // GD):
            pltpu.async_copy(
                src_ref.at[idxs[p].at[pl.ds(d * GD, GD)]],
                stages[p].at[pl.ds(d * GD, GD)], gsems[p])

    def drain_gather(p, src_ref):
        for d in range(G // GD):
            pltpu.make_async_copy(
                src_ref.at[idxs[p].at[pl.ds(d * GD, GD)]],
                stages[p].at[pl.ds(d * GD, GD)], gsems[p]).wait()

    def fire_scatter(p, k):
        for d in range(G // GD):
            pltpu.async_copy(stages[p].at[pl.ds(d * GD, GD)],
                             acc.at[row_v.at[k * (G // GD) + d]], ssems[p],
                             add=True)

    def drain_scatter(p, k):
        for d in range(G // GD):
            pltpu.make_async_copy(stages[p].at[pl.ds(d * GD, GD)],
                                  acc.at[row_v.at[k * (G // GD) + d]],
                                  ssems[p]).wait()

    def spmm_pass(src_ref, dst_ref, b):
        pltpu.sync_copy(z_ref, acc.at[pl.ds(s * RPT, RPT)])
        plsc.subcore_barrier()

        off = b * NP
        mk_idx(0, 0, off)
        fire_gather(0, src_ref)

        def blk(m, _):
            for u in range(NSTG):
                k = m * NSTG + u
                q = (u + 1) % NSTG

                @pl.when(k + 1 < NBLK)
                def _prefetch():
                    mk_idx(q, k + 1, off)

                    @pl.when(k >= 1)
                    def _drain_prev_scatter():
                        drain_scatter(q, k - 1)
                    fire_gather(q, src_ref)

                drain_gather(u, src_ref)
                scale(u, k * G)
                fire_scatter(u, k)
            return 0
        lax.fori_loop(0, NBLK // NSTG, blk, 0)
        for i in range(NSTG):
            kk = NBLK - NSTG + i
            drain_scatter(kk % NSTG, kk)
        plsc.subcore_barrier()
        pltpu.sync_copy(acc.at[pl.ds(s * RPT, RPT)],
                        dst_ref.at[pl.ds(b * NP + s * RPT, RPT)])

    for (ch, rh, vh, dst_a, dst_b) in (
            (c1_ref, r1_ref, v1_ref, y1_ref, y2_ref),
            (c2_ref, r2_ref, v2_ref, y3_ref, y4_ref)):
        pltpu.sync_copy(ch.at[s], col_v)
        pltpu.sync_copy(rh.at[s], row_v)
        pltpu.sync_copy(vh.at[s], val_v)

        def batch_body(bi, _):
            b = c * BPC + bi
            spmm_pass(x0_ref, dst_a, b)
            spmm_pass(dst_a, dst_b, b)
            return 0
        lax.fori_loop(0, BPC, batch_body, 0)


def _mm_body(x0_ref, y1_ref, y2_ref, y3_ref, y4_ref, w_ref, b_ref, o_ref):
    acc = jnp.dot(x0_ref[0], w_ref[0], preferred_element_type=jnp.float32)
    acc += jnp.dot(y1_ref[0], w_ref[1], preferred_element_type=jnp.float32)
    acc += jnp.dot(y2_ref[0], w_ref[2], preferred_element_type=jnp.float32)
    acc += jnp.dot(y3_ref[0], w_ref[3], preferred_element_type=jnp.float32)
    acc += jnp.dot(y4_ref[0], w_ref[4], preferred_element_type=jnp.float32)
    o_ref[0] = jnp.tanh(acc + b_ref[...])


def _prep_edges(col, row, val):
    cp = jnp.pad(col.reshape(NS, EPT), ((0, 0), (0, EPTP - EPT)))
    rp = jnp.pad(row.reshape(NS, EPT), ((0, 0), (0, EPTP - EPT)))
    vp = jnp.pad(val.reshape(NS, EPT), ((0, 0), (0, EPTP - EPT)))
    return cp, rp.reshape(NS, EPTP // GD, GD), vp


def kernel(inputs, state_t, s1_row, s1_col, s1_val, s2_row, s2_col, s2_val,
           weights, biases):
    Bb, Nn, in_dim = inputs.shape
    x_cat = jnp.concatenate([inputs, state_t], axis=2)
    in_size = x_cat.shape[2]
    x0p = jnp.pad(x_cat, ((0, 0), (0, NP - Nn), (0, PADW - in_size)))
    x0f = x0p.reshape(Bb * NP, PADW)
    zeros_hbm = jnp.zeros((RPT, PADW), jnp.float32)

    c1, r1, v1 = _prep_edges(s1_col, s1_row, s1_val)
    c2, r2, v2 = _prep_edges(s2_col, s2_row, s2_val)

    mesh = plsc.VectorSubcoreMesh(core_axis_name="c", subcore_axis_name="s")
    sc = pl.kernel(
        _sc_body,
        out_type=[jax.ShapeDtypeStruct((Bb * NP, PADW), jnp.float32)] * 4,
        mesh=mesh,
        compiler_params=pltpu.CompilerParams(use_tc_tiling_on_sc=False),
        scratch_types=[
            pltpu.VMEM((EPTP,), jnp.int32),
            pltpu.VMEM((EPTP // GD, GD), jnp.int32),
            pltpu.VMEM((EPTP,), jnp.float32),
            pltpu.VMEM((G,), jnp.int32),
            pltpu.VMEM((G,), jnp.int32),
            pltpu.VMEM((G, PADW), jnp.float32),
            pltpu.VMEM((G, PADW), jnp.float32),
            pltpu.VMEM_SHARED((NP, PADW), jnp.float32),
        ] + [pltpu.SemaphoreType.DMA] * 4,
    )
    y1, y2, y3, y4 = sc(x0f, c1, r1, v1, c2, r2, v2, zeros_hbm)

    wm = weights.reshape(in_size, 5, HID)
    wa = jnp.stack([wm[:, 0] - wm[:, 2] - wm[:, 4], wm[:, 1], 2.0 * wm[:, 2],
                    wm[:, 3], 2.0 * wm[:, 4]], axis=0)
    wp = jnp.pad(wa, ((0, 0), (0, PADW - in_size), (0, 0)))

    NB = 1000
    feat_spec = pl.BlockSpec((1, NB, PADW), lambda bb, nn: (bb, nn, 0))
    out = pl.pallas_call(
        _mm_body,
        grid=(Bb, Nn // NB),
        in_specs=[feat_spec] * 5 + [
            pl.BlockSpec((5, PADW, HID), lambda bb, nn: (0, 0, 0)),
            pl.BlockSpec((HID,), lambda bb, nn: (0,)),
        ],
        out_specs=pl.BlockSpec((1, NB, HID), lambda bb, nn: (bb, nn, 0)),
        out_shape=jax.ShapeDtypeStruct((Bb, Nn, HID), jnp.float32),
    )(x0p, y1.reshape(Bb, NP, PADW), y2.reshape(Bb, NP, PADW),
      y3.reshape(Bb, NP, PADW), y4.reshape(Bb, NP, PADW), wp, biases)
    return out


# speedup vs baseline: 1.3114x; 1.0211x over previous
import jax
import jax.numpy as jnp
from jax import lax
from jax.experimental import pallas as pl
from jax.experimental.pallas import tpu as pltpu
from jax.experimental.pallas import tpu_sc as plsc

N = 10000
NP = 10240
B = 16
HID = 64
PADW = 80
E = 160000
NC = 2
NS = 16
EPT = E // NS
EPTP = 10240
G = 128
NBLK = EPTP // G
NSTG = 4
RPT = NP // NS
BPC = B // NC
NVR = PADW // 16


def _sc_body(x0_ref, c1_ref, r1_ref, v1_ref, c2_ref, r2_ref, v2_ref, z_ref,
             y1_ref, y2_ref, y3_ref, y4_ref,
             col_v, row_v, val_v, idx0, idx1, idx2, idx3,
             st0, st1, st2, st3, acc,
             gsem0, gsem1, gsem2, gsem3, ssem0, ssem1, ssem2, ssem3):
    c = lax.axis_index("c")
    s = lax.axis_index("s")
    stages = (st0, st1, st2, st3)
    idxs = (idx0, idx1, idx2, idx3)
    gsems = (gsem0, gsem1, gsem2, gsem3)
    ssems = (ssem0, ssem1, ssem2, ssem3)

    def mk_idx(p, k, off):
        for i in range(G // 16):
            sl = pl.ds(i * 16, 16)
            idxs[p][sl] = col_v[pl.ds(k * G + i * 16, 16)] + off

    def scale(p, kG):
        st = stages[p]

        def grp(g, carry):
            chunk = val_v[pl.ds(carry + g * 16, 16)]
            for u in range(16):
                vv = jnp.broadcast_to(chunk[u], (16,))
                j = g * 16 + u
                for r in range(NVR):
                    st[j, pl.ds(r * 16, 16)] = st[j, pl.ds(r * 16, 16)] * vv
            return carry
        lax.fori_loop(0, G // 16, grp, kG)

    def spmm_pass(src_ref, dst_ref, b):
        pltpu.sync_copy(z_ref, acc.at[pl.ds(s * RPT, RPT)])
        plsc.subcore_barrier()

        off = b * NP
        mk_idx(0, 0, off)
        pltpu.async_copy(src_ref.at[idx0], st0, gsem0)
        mk_idx(1, 1, off)
        pltpu.async_copy(src_ref.at[idx1], st1, gsem1)

        def blk(m, _):
            for u in range(NSTG):
                k = m * NSTG + u
                w = (u + 2) % NSTG

                pltpu.make_async_copy(src_ref.at[idxs[u]], stages[u],
                                      gsems[u]).wait()
                scale(u, k * G)
                pltpu.async_copy(stages[u], acc.at[row_v.at[k]], ssems[u],
                                 add=True)

                @pl.when(k + 2 < NBLK)
                def _prefetch():
                    mk_idx(w, k + 2, off)

                    @pl.when(k >= 2)
                    def _drain_prev_scatter():
                        pltpu.make_async_copy(
                            stages[w], acc.at[row_v.at[k - 2]],
                            ssems[w]).wait()
                    pltpu.async_copy(src_ref.at[idxs[w]], stages[w],
                                     gsems[w])
            return 0
        lax.fori_loop(0, NBLK // NSTG, blk, 0)
        for i in range(NSTG):
            kk = NBLK - NSTG + i
            pltpu.make_async_copy(
                stages[kk % NSTG], acc.at[row_v.at[kk]],
                ssems[kk % NSTG]).wait()
        plsc.subcore_barrier()
        pltpu.sync_copy(acc.at[pl.ds(s * RPT, RPT)],
                        dst_ref.at[pl.ds(b * NP + s * RPT, RPT)])

    for (ch, rh, vh, dst_a, dst_b) in (
            (c1_ref, r1_ref, v1_ref, y1_ref, y2_ref),
            (c2_ref, r2_ref, v2_ref, y3_ref, y4_ref)):
        pltpu.sync_copy(ch.at[s], col_v)
        pltpu.sync_copy(rh.at[s], row_v)
        pltpu.sync_copy(vh.at[s], val_v)

        def batch_body(bi, _):
            b = c * BPC + bi
            spmm_pass(x0_ref, dst_a, b)
            spmm_pass(dst_a, dst_b, b)
            return 0
        lax.fori_loop(0, BPC, batch_body, 0)


def _mm_body(x0_ref, y1_ref, y2_ref, y3_ref, y4_ref, w_ref, b_ref, o_ref):
    acc = jnp.dot(x0_ref[0], w_ref[0], preferred_element_type=jnp.float32)
    acc += jnp.dot(y1_ref[0], w_ref[1], preferred_element_type=jnp.float32)
    acc += jnp.dot(y2_ref[0], w_ref[2], preferred_element_type=jnp.float32)
    acc += jnp.dot(y3_ref[0], w_ref[3], preferred_element_type=jnp.float32)
    acc += jnp.dot(y4_ref[0], w_ref[4], preferred_element_type=jnp.float32)
    o_ref[0] = jnp.tanh(acc + b_ref[...])


def _prep_edges(col, row, val):
    cp = jnp.pad(col.reshape(NS, EPT), ((0, 0), (0, EPTP - EPT)))
    rp = jnp.pad(row.reshape(NS, EPT), ((0, 0), (0, EPTP - EPT)))
    vp = jnp.pad(val.reshape(NS, EPT), ((0, 0), (0, EPTP - EPT)))
    return cp, rp.reshape(NS, NBLK, G), vp


def kernel(inputs, state_t, s1_row, s1_col, s1_val, s2_row, s2_col, s2_val,
           weights, biases):
    Bb, Nn, in_dim = inputs.shape
    x_cat = jnp.concatenate([inputs, state_t], axis=2)
    in_size = x_cat.shape[2]
    x0p = jnp.pad(x_cat, ((0, 0), (0, NP - Nn), (0, PADW - in_size)))
    x0f = x0p.reshape(Bb * NP, PADW)
    zeros_hbm = jnp.zeros((RPT, PADW), jnp.float32)

    c1, r1, v1 = _prep_edges(s1_col, s1_row, s1_val)
    c2, r2, v2 = _prep_edges(s2_col, s2_row, s2_val)

    mesh = plsc.VectorSubcoreMesh(core_axis_name="c", subcore_axis_name="s")
    sc = pl.kernel(
        _sc_body,
        out_type=[jax.ShapeDtypeStruct((Bb * NP, PADW), jnp.float32)] * 4,
        mesh=mesh,
        compiler_params=pltpu.CompilerParams(use_tc_tiling_on_sc=False),
        scratch_types=[
            pltpu.VMEM((EPTP,), jnp.int32),
            pltpu.VMEM((NBLK, G), jnp.int32),
            pltpu.VMEM((EPTP,), jnp.float32),
            pltpu.VMEM((G,), jnp.int32),
            pltpu.VMEM((G,), jnp.int32),
            pltpu.VMEM((G,), jnp.int32),
            pltpu.VMEM((G,), jnp.int32),
            pltpu.VMEM((G, PADW), jnp.float32),
            pltpu.VMEM((G, PADW), jnp.float32),
            pltpu.VMEM((G, PADW), jnp.float32),
            pltpu.VMEM((G, PADW), jnp.float32),
            pltpu.VMEM_SHARED((NP, PADW), jnp.float32),
        ] + [pltpu.SemaphoreType.DMA] * 8,
    )
    y1, y2, y3, y4 = sc(x0f, c1, r1, v1, c2, r2, v2, zeros_hbm)

    wm = weights.reshape(in_size, 5, HID)
    wa = jnp.stack([wm[:, 0] - wm[:, 2] - wm[:, 4], wm[:, 1], 2.0 * wm[:, 2],
                    wm[:, 3], 2.0 * wm[:, 4]], axis=0)
    wp = jnp.pad(wa, ((0, 0), (0, PADW - in_size), (0, 0)))

    NB = 1000
    feat_spec = pl.BlockSpec((1, NB, PADW), lambda bb, nn: (bb, nn, 0))
    out = pl.pallas_call(
        _mm_body,
        grid=(Bb, Nn // NB),
        in_specs=[feat_spec] * 5 + [
            pl.BlockSpec((5, PADW, HID), lambda bb, nn: (0, 0, 0)),
            pl.BlockSpec((HID,), lambda bb, nn: (0,)),
        ],
        out_specs=pl.BlockSpec((1, NB, HID), lambda bb, nn: (bb, nn, 0)),
        out_shape=jax.ShapeDtypeStruct((Bb, Nn, HID), jnp.float32),
    )(x0p, y1.reshape(Bb, NP, PADW), y2.reshape(Bb, NP, PADW),
      y3.reshape(Bb, NP, PADW), y4.reshape(Bb, NP, PADW), wp, biases)
    return out
